# baseline (speedup 1.0000x reference)
import functools

import jax
import jax.numpy as jnp
from jax import lax
from jax.experimental import pallas as pl
from jax.experimental.pallas import tpu as pltpu
from jax.experimental.pallas import tpu_sc as plsc


def _tc_body(nsb, nch, pool, x_ref, w_ref, b_ref, logits_ref, idx2_ref, acc_ref):
    bi = pl.program_id(0)
    ci = pl.program_id(1)
    m = jnp.max(x_ref[...], axis=0, keepdims=True)

    @pl.when(ci == 0)
    def _init():
        acc_ref[pl.ds(bi, 1), :] = m

    @pl.when(ci > 0)
    def _acc():
        acc_ref[pl.ds(bi, 1), :] = jnp.maximum(acc_ref[pl.ds(bi, 1), :], m)

    B = logits_ref.shape[0]

    @pl.when((bi == B - 1) & (ci == nsb - 1))
    def _final():
        xmax = acc_ref[...]
        logits = lax.dot_general(
            xmax, w_ref[...], (((1,), (1,)), ((), ())),
            preferred_element_type=jnp.float32,
        ) + b_ref[...]
        logits_ref[...] = logits
        idx = jnp.argmax(logits, axis=1).astype(jnp.int32)
        j = lax.broadcasted_iota(jnp.int32, idx2_ref.shape, 1)
        k = j // nch
        c = j - k * nch
        idx2_ref[...] = (k * pool + idx[:, None]) * nch + c


def _tc_reduce_classify(x_embed, W, b, nch):
    B, S, E = x_embed.shape
    P = W.shape[0]
    S_BLK = 512
    nsb = S // S_BLK
    x2 = x_embed.reshape(B * S, E)
    return pl.pallas_call(
        functools.partial(_tc_body, nsb, nch, P),
        grid=(B, nsb),
        in_specs=[
            pl.BlockSpec((S_BLK, E), lambda bi, ci: (bi * nsb + ci, 0)),
            pl.BlockSpec((P, E), lambda bi, ci: (0, 0)),
            pl.BlockSpec((1, P), lambda bi, ci: (0, 0)),
        ],
        out_specs=[
            pl.BlockSpec((B, P), lambda bi, ci: (0, 0)),
            pl.BlockSpec((B, 2 * nch), lambda bi, ci: (0, 0)),
        ],
        out_shape=[
            jax.ShapeDtypeStruct((B, P), jnp.float32),
            jax.ShapeDtypeStruct((B, 2 * nch), jnp.int32),
        ],
        scratch_shapes=[pltpu.VMEM((B, E), jnp.float32)],
        compiler_params=pltpu.CompilerParams(
            dimension_semantics=("arbitrary", "arbitrary")),
    )(x2, W, b.reshape(1, P))


def _sc_gather(table, idx_flat, n_workers, nc):
    n_rows, dc = idx_flat.shape[0], table.shape[1]
    b_per_w = n_rows // n_workers
    mesh = plsc.VectorSubcoreMesh(core_axis_name="c", subcore_axis_name="s")

    @functools.partial(
        pl.kernel, mesh=mesh,
        out_type=jax.ShapeDtypeStruct((n_rows, dc), jnp.float32),
        scratch_types=[
            pltpu.VMEM((b_per_w,), jnp.int32),
            pltpu.VMEM((b_per_w, dc), jnp.float32),
            pltpu.SemaphoreType.DMA,
        ],
    )
    def _body(table_hbm, idx_hbm, out_hbm, idx_v, rows_v, sem):
        wid = lax.axis_index("s") * nc + lax.axis_index("c")
        base = wid * b_per_w
        pltpu.sync_copy(idx_hbm.at[pl.ds(base, b_per_w)], idx_v)
        pltpu.async_copy(table_hbm.at[idx_v], rows_v, sem).wait()
        pltpu.sync_copy(rows_v, out_hbm.at[pl.ds(base, b_per_w)])

    return _body(table, idx_flat)


def kernel(x_embed, prompt, W, b):
    B, S, E = x_embed.shape
    NL, TWO, P, L, H, D = prompt.shape
    info = plsc.get_sparse_core_info()
    n_workers = info.num_cores * info.num_subcores
    nch = (8 * n_workers) // (B * TWO)
    dc = (L * H * D) // nch
    logits, idx2 = _tc_reduce_classify(x_embed, W, b, nch)
    table = prompt.reshape(TWO * P * nch, dc)
    out2 = _sc_gather(table, idx2.reshape(-1), n_workers, info.num_cores)
    e_prompt = out2.reshape(B, TWO, L, H, D)[None]
    return logits, e_prompt
